# baseline (speedup 1.0000x reference)
import functools

import jax
import jax.numpy as jnp
import numpy as np
from jax import lax
from jax.experimental import pallas as pl
from jax.experimental.pallas import tpu as pltpu
from jax.experimental.pallas import tpu_sc as plsc

N_ROWS = 3300001
FEAT = 16
NTYPES = 13
B, P, S = 8192, 16, 9

NC, NS, LANES = 2, 16, 16
NW = NC * NS
PAIRS_W = B // NW
PATHS_W = PAIRS_W * P
STEPS_W = PATHS_W * S
CHUNK = 128
CHUNKS = STEPS_W // CHUNK
WINDOW = 32

RB = 32768
NB = -(-N_ROWS // RB)
RBF = RB * FEAT
UROWS = RB // 8
NT8 = NTYPES * 8


def _row_scores_body(vff_ref, vcq_ref, e13_ref, rep8_ref, e8c_ref, tv_ref,
                     w2v_ref, out_ref):
    vfp = vff_ref[...].reshape(UROWS, 128)
    s = jnp.dot(vfp, e13_ref[...], preferred_element_type=jnp.float32)
    s = s + w2v_ref[0:1, :]
    rep = jnp.dot(vcq_ref[...], rep8_ref[...], preferred_element_type=jnp.float32)
    sel = jnp.where(rep == tv_ref[0:1, :], s, 0.0)
    out_ref[...] = jnp.dot(sel, e8c_ref[...], preferred_element_type=jnp.float32)


def _row_scores(Vf, Vc, W1, W2):
    j = np.arange(NT8)
    l = np.arange(128)
    mask13 = ((l[:, None] // FEAT) == (j[None, :] // NTYPES)).astype(np.float32)
    jmod = jnp.asarray(j % NTYPES)
    e13 = jnp.tile(W1.T[:, jmod], (8, 1)) * mask13
    rep8 = jnp.asarray((j[None, :] // NTYPES == np.arange(8)[:, None])
                       .astype(np.float32))
    e8c = rep8.T
    tv = jnp.asarray(np.broadcast_to((j % NTYPES).astype(np.float32),
                                     (8, NT8)))
    w2v = jnp.broadcast_to(W2[:, 0][jmod], (8, NT8))

    vff = Vf.reshape(N_ROWS * FEAT)
    vcq = jnp.pad(Vc, (0, NB * RB - N_ROWS)).astype(jnp.float32) \
             .reshape(NB * UROWS, 8)
    out = pl.pallas_call(
        _row_scores_body,
        grid=(NB,),
        in_specs=[
            pl.BlockSpec((RBF,), lambda i: (i,)),
            pl.BlockSpec((UROWS, 8), lambda i: (i, 0)),
            pl.BlockSpec((128, NT8), lambda i: (0, 0)),
            pl.BlockSpec((8, NT8), lambda i: (0, 0)),
            pl.BlockSpec((NT8, 8), lambda i: (0, 0)),
            pl.BlockSpec((8, NT8), lambda i: (0, 0)),
            pl.BlockSpec((8, NT8), lambda i: (0, 0)),
        ],
        out_specs=pl.BlockSpec((UROWS, 8), lambda i: (i, 0)),
        out_shape=jax.ShapeDtypeStruct((NB * UROWS, 8), jnp.float32),
    )(vff, vcq, e13, rep8, e8c, tv, w2v)
    r = out.reshape(NB * RB)
    c_last = Vc[N_ROWS - 1]
    r_last = jnp.dot(W1[c_last], Vf[N_ROWS - 1]) + W2[c_last, 0]
    return r.at[N_ROWS - 1].set(r_last)


def _sc_body(steps_hbm, r_hbm, b_hbm, out_hbm,
             idx_v, vals_v, acc_v, out_v, b_v, sem):
    w = lax.axis_index("s") * NC + lax.axis_index("c")

    pltpu.sync_copy(steps_hbm.at[w], idx_v)
    pltpu.sync_copy(b_hbm, b_v)

    def mk(i):
        return pltpu.make_async_copy(
            r_hbm.at[idx_v.at[i]], vals_v.at[pl.ds(i * CHUNK, CHUNK)], sem)

    def fire(i, c):
        mk(i).start()
        return c

    def roll(i, c):
        mk(i).start()
        mk(i - WINDOW).wait()
        return c

    def drain(i, c):
        mk(i).wait()
        return c

    lax.fori_loop(0, WINDOW, fire, 0)
    lax.fori_loop(WINDOW, CHUNKS, roll, 0)
    lax.fori_loop(CHUNKS - WINDOW, CHUNKS, drain, 0)

    iota = lax.iota(jnp.int32, LANES)

    def psum(g, c):
        base = g * LANES
        flat0 = (base + iota) * S
        acc = plsc.load_gather(vals_v, [flat0])
        for s in range(1, S):
            acc = acc + plsc.load_gather(vals_v, [flat0 + s])
        acc_v[pl.ds(base, LANES)] = acc
        return c

    lax.fori_loop(0, PATHS_W // LANES, psum, 0)

    def pmax(g, c):
        base = g * (LANES * P)
        m = plsc.load_gather(acc_v, [base + iota * P])
        for j in range(1, P):
            m = jnp.maximum(m, plsc.load_gather(acc_v, [base + iota * P + j]))
        z = m + b_v[...]
        out_v[pl.ds(g * LANES, LANES)] = 1.0 / (1.0 + jnp.exp(-z))
        return c

    lax.fori_loop(0, PAIRS_W // LANES, pmax, 0)

    pltpu.sync_copy(out_v, out_hbm.at[pl.ds(w * PAIRS_W, PAIRS_W)])


def _sc_reduce_fn():
    return pl.kernel(
        _sc_body,
        out_type=jax.ShapeDtypeStruct((B,), jnp.float32),
        mesh=plsc.VectorSubcoreMesh(
            core_axis_name="c", subcore_axis_name="s",
            num_cores=NC, num_subcores=NS),
        scratch_types=[
            pltpu.VMEM((CHUNKS, CHUNK), jnp.int32),
            pltpu.VMEM((STEPS_W,), jnp.float32),
            pltpu.VMEM((PATHS_W,), jnp.float32),
            pltpu.VMEM((PAIRS_W,), jnp.float32),
            pltpu.VMEM((LANES,), jnp.float32),
            pltpu.SemaphoreType.DMA,
        ],
        compiler_params=pltpu.CompilerParams(needs_layout_passes=False),
    )


def kernel(all_steps, Vf, Vc, W1, W2, b):
    r = _row_scores(Vf, Vc.astype(jnp.int32), W1, W2)
    steps = all_steps.astype(jnp.int32).reshape(NW, CHUNKS, CHUNK)
    b16 = jnp.broadcast_to(b.astype(jnp.float32), (LANES,))
    return _sc_reduce_fn()(steps, r, b16)


# speedup vs baseline: 1.0990x; 1.0990x over previous
import jax
import jax.numpy as jnp
from jax import lax
from jax.experimental import pallas as pl
from jax.experimental.pallas import tpu as pltpu
from jax.experimental.pallas import tpu_sc as plsc

N_ROWS = 3300001
FEAT = 16
NTYPES = 13
B, P, S = 8192, 16, 9

NC, NS, LANES = 2, 16, 16
NW = NC * NS
PAIRS_W = B // NW
PATHS_W = PAIRS_W * P
STEPS_W = PATHS_W * S
CHUNK = 128
CHUNKS = STEPS_W // CHUNK

GPC = 9
CH_STEPS = GPC * CHUNK
NCHUNK = STEPS_W // CH_STEPS
GROUPS = CH_STEPS // LANES
PPC = CH_STEPS // S


def _sc_body(steps_hbm, vf_hbm, vc_hbm, w1_hbm, w2_hbm, b_hbm, out_hbm,
             idx_v, rows_v, cats_v, w1_v, w2_v, svals_v, acc_v, out_v, b_v,
             semr, semc):
    w = lax.axis_index("s") * NC + lax.axis_index("c")

    pltpu.sync_copy(steps_hbm.at[w], idx_v)
    pltpu.sync_copy(w1_hbm, w1_v)
    pltpu.sync_copy(w2_hbm, w2_v)
    pltpu.sync_copy(b_hbm, b_v)

    iota = lax.iota(jnp.int32, LANES)

    def mk_rows(c, k):
        return pltpu.make_async_copy(
            vf_hbm.at[idx_v.at[c * GPC + k]],
            rows_v.at[pl.ds(((c % 2) * GPC + k) * CHUNK, CHUNK), :],
            semr)

    def mk_cats(c, k):
        return pltpu.make_async_copy(
            vc_hbm.at[idx_v.at[c * GPC + k]],
            cats_v.at[pl.ds((c % 2) * CH_STEPS + k * CHUNK, CHUNK)],
            semc)

    def fire(c):
        for k in range(GPC):
            mk_rows(c, k).start()
            mk_cats(c, k).start()

    def drain(c):
        for k in range(GPC):
            mk_rows(c, k).wait()
            mk_cats(c, k).wait()

    fire(0)

    def chunk_body(c, carry):
        @pl.when(c + 1 < NCHUNK)
        def _():
            fire(c + 1)
        drain(c)
        buf = c % 2
        rbase = buf * CH_STEPS
        cbase = buf * CH_STEPS

        def group(g, carry2):
            sbase = g * LANES
            cats = cats_v[pl.ds(cbase + sbase, LANES)]
            rowi = rbase + sbase + iota
            zero = jnp.zeros((LANES,), jnp.int32)
            acc = plsc.load_gather(w2_v, [cats, zero])
            for f in range(FEAT):
                fv = plsc.load_gather(rows_v, [rowi, zero + f])
                wv = plsc.load_gather(w1_v, [cats, zero + f])
                acc = acc + fv * wv
            svals_v[pl.ds(sbase, LANES)] = acc
            return carry2

        lax.fori_loop(0, GROUPS, group, 0)

        def psum(g, carry2):
            flat0 = (g * LANES + iota) * S
            a = plsc.load_gather(svals_v, [flat0])
            for s in range(1, S):
                a = a + plsc.load_gather(svals_v, [flat0 + s])
            acc_v[pl.ds(c * PPC + g * LANES, LANES)] = a
            return carry2

        lax.fori_loop(0, PPC // LANES, psum, 0)
        return carry

    lax.fori_loop(0, NCHUNK, chunk_body, 0)

    def pmax(g, carry):
        base = g * (LANES * P)
        m = plsc.load_gather(acc_v, [base + iota * P])
        for j in range(1, P):
            m = jnp.maximum(m, plsc.load_gather(acc_v, [base + iota * P + j]))
        z = m + b_v[...]
        out_v[pl.ds(g * LANES, LANES)] = 1.0 / (1.0 + jnp.exp(-z))
        return carry

    lax.fori_loop(0, PAIRS_W // LANES, pmax, 0)

    pltpu.sync_copy(out_v, out_hbm.at[pl.ds(w * PAIRS_W, PAIRS_W)])


def _sc_kernel_fn():
    return pl.kernel(
        _sc_body,
        out_type=jax.ShapeDtypeStruct((B,), jnp.float32),
        mesh=plsc.VectorSubcoreMesh(
            core_axis_name="c", subcore_axis_name="s",
            num_cores=NC, num_subcores=NS),
        scratch_types=[
            pltpu.VMEM((CHUNKS, CHUNK), jnp.int32),
            pltpu.VMEM((2 * CH_STEPS, FEAT), jnp.float32),
            pltpu.VMEM((2 * CH_STEPS,), jnp.int32),
            pltpu.VMEM((NTYPES, FEAT), jnp.float32),
            pltpu.VMEM((NTYPES, 1), jnp.float32),
            pltpu.VMEM((CH_STEPS,), jnp.float32),
            pltpu.VMEM((PATHS_W,), jnp.float32),
            pltpu.VMEM((PAIRS_W,), jnp.float32),
            pltpu.VMEM((LANES,), jnp.float32),
            pltpu.SemaphoreType.DMA,
            pltpu.SemaphoreType.DMA,
        ],
        compiler_params=pltpu.CompilerParams(needs_layout_passes=False,
                                             use_tc_tiling_on_sc=False),
    )


def kernel(all_steps, Vf, Vc, W1, W2, b):
    steps = all_steps.astype(jnp.int32).reshape(NW, CHUNKS, CHUNK)
    b16 = jnp.broadcast_to(b.astype(jnp.float32), (LANES,))
    return _sc_kernel_fn()(steps, Vf, Vc.astype(jnp.int32), W1, W2, b16)


# speedup vs baseline: 5.9264x; 5.3926x over previous
import functools

import jax
import jax.numpy as jnp
import numpy as np
from jax import lax
from jax.experimental import pallas as pl
from jax.experimental.pallas import tpu as pltpu
from jax.experimental.pallas import tpu_sc as plsc

N_ROWS = 3300001
FEAT = 16
NTYPES = 13
B, P, S = 8192, 16, 9

NC, NS, LANES = 2, 16, 16
NW = NC * NS
PAIRS_W = B // NW
PATHS_W = PAIRS_W * P
STEPS_W = PATHS_W * S
CHUNK = 128
CHUNKS = STEPS_W // CHUNK
WINDOW = 32

RB = 32768
NB = -(-N_ROWS // RB)


def _row_scores_body(vft_ref, vc_ref, w1p_ref, w2c_ref, out_ref):
    s = jnp.dot(w1p_ref[...], vft_ref[...],
                preferred_element_type=jnp.float32)
    s = s + w2c_ref[:, 0:1]
    cats = vc_ref[0]
    tid = lax.broadcasted_iota(jnp.int32, s.shape, 0)
    out_ref[0] = jnp.sum(jnp.where(tid == cats, s, 0.0), axis=0,
                         keepdims=True)


def _row_scores(Vf, Vc, W1, W2):
    w1p = jnp.pad(W1, ((0, 16 - NTYPES), (0, 0)))
    w2c = jnp.broadcast_to(jnp.pad(W2, ((0, 16 - NTYPES), (0, 0))),
                           (16, 128))
    vcl = jnp.pad(Vc, (0, NB * RB - N_ROWS)).reshape(NB, 1, RB)
    vft = Vf.T
    out = pl.pallas_call(
        _row_scores_body,
        grid=(NB,),
        in_specs=[
            pl.BlockSpec((FEAT, RB), lambda i: (0, i)),
            pl.BlockSpec((1, 1, RB), lambda i: (i, 0, 0)),
            pl.BlockSpec((16, FEAT), lambda i: (0, 0)),
            pl.BlockSpec((16, 128), lambda i: (0, 0)),
        ],
        out_specs=pl.BlockSpec((1, 1, RB), lambda i: (i, 0, 0)),
        out_shape=jax.ShapeDtypeStruct((NB, 1, RB), jnp.float32),
    )(vft, vcl, w1p, w2c)
    return out.reshape(NB * RB)


def _sc_body(steps_hbm, r_hbm, b_hbm, out_hbm,
             idx_v, vals_v, acc_v, out_v, b_v, sem):
    w = lax.axis_index("s") * NC + lax.axis_index("c")

    pltpu.sync_copy(steps_hbm.at[w], idx_v)
    pltpu.sync_copy(b_hbm, b_v)

    def mk(i):
        return pltpu.make_async_copy(
            r_hbm.at[idx_v.at[i]], vals_v.at[pl.ds(i * CHUNK, CHUNK)], sem)

    def fire(i, c):
        mk(i).start()
        return c

    def roll(i, c):
        mk(i).start()
        mk(i - WINDOW).wait()
        return c

    def drain(i, c):
        mk(i).wait()
        return c

    lax.fori_loop(0, WINDOW, fire, 0)
    lax.fori_loop(WINDOW, CHUNKS, roll, 0)
    lax.fori_loop(CHUNKS - WINDOW, CHUNKS, drain, 0)

    iota = lax.iota(jnp.int32, LANES)

    def psum(g, c):
        base = g * LANES
        flat0 = (base + iota) * S
        acc = plsc.load_gather(vals_v, [flat0])
        for s in range(1, S):
            acc = acc + plsc.load_gather(vals_v, [flat0 + s])
        acc_v[pl.ds(base, LANES)] = acc
        return c

    lax.fori_loop(0, PATHS_W // LANES, psum, 0)

    def pmax(g, c):
        base = g * (LANES * P)
        m = plsc.load_gather(acc_v, [base + iota * P])
        for j in range(1, P):
            m = jnp.maximum(m, plsc.load_gather(acc_v, [base + iota * P + j]))
        z = m + b_v[...]
        out_v[pl.ds(g * LANES, LANES)] = 1.0 / (1.0 + jnp.exp(-z))
        return c

    lax.fori_loop(0, PAIRS_W // LANES, pmax, 0)

    pltpu.sync_copy(out_v, out_hbm.at[pl.ds(w * PAIRS_W, PAIRS_W)])


def _sc_reduce_fn():
    return pl.kernel(
        _sc_body,
        out_type=jax.ShapeDtypeStruct((B,), jnp.float32),
        mesh=plsc.VectorSubcoreMesh(
            core_axis_name="c", subcore_axis_name="s",
            num_cores=NC, num_subcores=NS),
        scratch_types=[
            pltpu.VMEM((CHUNKS, CHUNK), jnp.int32),
            pltpu.VMEM((STEPS_W,), jnp.float32),
            pltpu.VMEM((PATHS_W,), jnp.float32),
            pltpu.VMEM((PAIRS_W,), jnp.float32),
            pltpu.VMEM((LANES,), jnp.float32),
            pltpu.SemaphoreType.DMA,
        ],
        compiler_params=pltpu.CompilerParams(needs_layout_passes=False),
    )


def kernel(all_steps, Vf, Vc, W1, W2, b):
    r = _row_scores(Vf, Vc.astype(jnp.int32), W1, W2)
    steps = all_steps.astype(jnp.int32).reshape(NW, CHUNKS, CHUNK)
    b16 = jnp.broadcast_to(b.astype(jnp.float32), (LANES,))
    return _sc_reduce_fn()(steps, r, b16)


# speedup vs baseline: 5.9307x; 1.0007x over previous
import jax
import jax.numpy as jnp
from jax import lax
from jax.experimental import pallas as pl
from jax.experimental.pallas import tpu as pltpu
from jax.experimental.pallas import tpu_sc as plsc

N_ROWS = 3300001
FEAT = 16
NTYPES = 13
B, P, S = 8192, 16, 9

NC, NS, LANES = 2, 16, 16
NW = NC * NS
PAIRS_W = B // NW
PATHS_W = PAIRS_W * P
STEPS_W = PATHS_W * S
CHUNK = 128
CHUNKS = STEPS_W // CHUNK
WINDOW = 32

RB = 32768
NB = -(-N_ROWS // RB)


def _row_scores_body(vft_ref, vc_ref, w1p_ref, w2c_ref, out_ref):
    s = jnp.dot(w1p_ref[...], vft_ref[...],
                preferred_element_type=jnp.float32)
    s = s + w2c_ref[:, 0:1]
    cats = vc_ref[0]
    tid = lax.broadcasted_iota(jnp.int32, s.shape, 0)
    out_ref[0] = jnp.sum(jnp.where(tid == cats, s, 0.0), axis=0,
                         keepdims=True)


def _row_scores(Vf, Vc, W1, W2):
    w1p = jnp.pad(W1, ((0, 16 - NTYPES), (0, 0)))
    w2c = jnp.broadcast_to(jnp.pad(W2, ((0, 16 - NTYPES), (0, 0))),
                           (16, 128))
    vcl = jnp.pad(Vc, (0, NB * RB - N_ROWS)).reshape(NB, 1, RB)
    vft = Vf.T
    out = pl.pallas_call(
        _row_scores_body,
        grid=(NB,),
        in_specs=[
            pl.BlockSpec((FEAT, RB), lambda i: (0, i)),
            pl.BlockSpec((1, 1, RB), lambda i: (i, 0, 0)),
            pl.BlockSpec((16, FEAT), lambda i: (0, 0)),
            pl.BlockSpec((16, 128), lambda i: (0, 0)),
        ],
        out_specs=pl.BlockSpec((1, 1, RB), lambda i: (i, 0, 0)),
        out_shape=jax.ShapeDtypeStruct((NB, 1, RB), jnp.float32),
    )(vft, vcl, w1p, w2c)
    return out.reshape(NB * RB)


def _sc_body(steps_hbm, r_hbm, b_hbm, out_hbm,
             idx_v, vals_v, acc_v, out_v, b_v, sem):
    w = lax.axis_index("s") * NC + lax.axis_index("c")

    pltpu.sync_copy(steps_hbm.at[w], idx_v)
    pltpu.sync_copy(b_hbm, b_v)

    def mk(i):
        return pltpu.make_async_copy(
            r_hbm.at[idx_v.at[i]], vals_v.at[pl.ds(i * CHUNK, CHUNK)], sem)

    def fire(i, c):
        mk(i).start()
        return c

    def roll(i, c):
        mk(i).start()
        mk(i - WINDOW).wait()
        return c

    def drain(i, c):
        mk(i).wait()
        return c

    lax.fori_loop(0, WINDOW, fire, 0)
    lax.fori_loop(WINDOW, CHUNKS, roll, 0)
    lax.fori_loop(CHUNKS - WINDOW, CHUNKS, drain, 0)

    iota = lax.iota(jnp.int32, LANES)

    def psum(g, c):
        base = g * LANES
        flat0 = (base + iota) * S
        acc = plsc.load_gather(vals_v, [flat0])
        for s in range(1, S):
            acc = acc + plsc.load_gather(vals_v, [flat0 + s])
        acc_v[pl.ds(base, LANES)] = acc
        return c

    lax.fori_loop(0, PATHS_W // LANES, psum, 0)

    def pmax(g, c):
        base = g * (LANES * P)
        m = plsc.load_gather(acc_v, [base + iota * P])
        for j in range(1, P):
            m = jnp.maximum(m, plsc.load_gather(acc_v, [base + iota * P + j]))
        z = m + b_v[...]
        out_v[pl.ds(g * LANES, LANES)] = 1.0 / (1.0 + jnp.exp(-z))
        return c

    lax.fori_loop(0, PAIRS_W // LANES, pmax, 0)

    pltpu.sync_copy(out_v, out_hbm.at[pl.ds(w * PAIRS_W, PAIRS_W)])


def _sc_reduce_fn():
    return pl.kernel(
        _sc_body,
        out_type=jax.ShapeDtypeStruct((B,), jnp.float32),
        mesh=plsc.VectorSubcoreMesh(
            core_axis_name="c", subcore_axis_name="s",
            num_cores=NC, num_subcores=NS),
        scratch_types=[
            pltpu.VMEM((CHUNKS, CHUNK), jnp.int32),
            pltpu.VMEM((STEPS_W,), jnp.float32),
            pltpu.VMEM((PATHS_W,), jnp.float32),
            pltpu.VMEM((PAIRS_W,), jnp.float32),
            pltpu.VMEM((LANES,), jnp.float32),
            pltpu.SemaphoreType.DMA,
        ],
        compiler_params=pltpu.CompilerParams(needs_layout_passes=False),
    )


def kernel(all_steps, Vf, Vc, W1, W2, b):
    r = _row_scores(Vf, Vc.astype(jnp.int32), W1, W2)
    steps = all_steps.astype(jnp.int32).reshape(NW, CHUNKS, CHUNK)
    b16 = jnp.broadcast_to(b.astype(jnp.float32), (LANES,))
    return _sc_reduce_fn()(steps, r, b16)
